# Initial kernel scaffold; baseline (speedup 1.0000x reference)
import jax
import jax.numpy as jnp
from jax.experimental import pallas as pl


def kernel(x, adj, pool):
    raise NotImplementedError("write your pallas kernel here")


# speedup vs baseline: 15.7425x; 15.7425x over previous
import functools

import jax
import jax.numpy as jnp
from jax import lax
from jax.experimental import pallas as pl
from jax.experimental.pallas import tpu as pltpu
from jax.experimental.pallas import tpu_sc as plsc

N = 10000
E = 320000
D = 128
NPAD = 10240
NC = 2
NS = 16
W = NC * NS
EW = E // W
C = 200
NCHUNK = EW // C
RS = NPAD // NS
DEGC = 2000

_mesh = plsc.VectorSubcoreMesh(core_axis_name="c", subcore_axis_name="s")


@functools.partial(
    pl.kernel,
    out_type=jax.ShapeDtypeStruct((W, NPAD), jnp.float32),
    mesh=_mesh,
    compiler_params=pltpu.CompilerParams(needs_layout_passes=False),
    scratch_types=[
        pltpu.VMEM((NPAD,), jnp.float32),
        pltpu.VMEM((DEGC,), jnp.int32),
    ],
)
def _deg_kernel(col_hbm, degparts_hbm, degbuf, colbuf):
    cid = lax.axis_index("c")
    sid = lax.axis_index("s")
    w = sid * NC + cid

    def zero(i, carry):
        degbuf[pl.ds(i * 16, 16)] = jnp.zeros((16,), jnp.float32)
        return carry

    lax.fori_loop(0, NPAD // 16, zero, None)

    ones = jnp.ones((16,), jnp.float32)

    def chunk(k, carry):
        pltpu.sync_copy(col_hbm.at[pl.ds(w * EW + k * DEGC, DEGC)], colbuf)

        def scat(j, c2):
            idx = colbuf[pl.ds(j * 16, 16)]
            plsc.addupdate_scatter(degbuf, [idx], ones)
            return c2

        lax.fori_loop(0, DEGC // 16, scat, None)
        return carry

    lax.fori_loop(0, EW // DEGC, chunk, None)
    pltpu.sync_copy(degbuf, degparts_hbm.at[w])


@functools.partial(
    pl.kernel,
    out_type=[
        jax.ShapeDtypeStruct((NPAD, D), jnp.float32),
        jax.ShapeDtypeStruct((NPAD, D), jnp.float32),
    ],
    mesh=_mesh,
    compiler_params=pltpu.CompilerParams(needs_layout_passes=False),
    scratch_types=[
        pltpu.VMEM_SHARED((NPAD, D), jnp.float32),
        pltpu.VMEM((C, D), jnp.float32),
        pltpu.VMEM((C,), jnp.int32),
        pltpu.VMEM((C,), jnp.int32),
        pltpu.SemaphoreType.DMA,
    ],
)
def _prop_kernel(g_hbm, row_hbm, col_hbm, zeros_hbm,
                 p0_hbm, p1_hbm,
                 acc_sh, rows_a, ir_a, ic_a, sem):
    cid = lax.axis_index("c")
    sid = lax.axis_index("s")
    w = sid * NC + cid
    rbase = sid * RS

    pltpu.sync_copy(zeros_hbm, acc_sh.at[pl.ds(rbase, RS)])
    plsc.subcore_barrier()

    ebase = w * EW

    def chunk(k, carry):
        eb = ebase + k * C
        pltpu.sync_copy(row_hbm.at[pl.ds(eb, C)], ir_a)
        pltpu.sync_copy(col_hbm.at[pl.ds(eb, C)], ic_a)
        pltpu.async_copy(g_hbm.at[ir_a], rows_a, sem).wait()
        pltpu.sync_copy(rows_a, acc_sh.at[ic_a], add=True)
        return carry

    lax.fori_loop(0, NCHUNK, chunk, None)
    plsc.subcore_barrier()

    @pl.when(cid == 0)
    def _():
        pltpu.sync_copy(acc_sh.at[pl.ds(rbase, RS)], p0_hbm.at[pl.ds(rbase, RS)])

    @pl.when(cid == 1)
    def _():
        pltpu.sync_copy(acc_sh.at[pl.ds(rbase, RS)], p1_hbm.at[pl.ds(rbase, RS)])


def _dis_body(dp_ref, dis_ref):
    s = jnp.sum(dp_ref[...], axis=0, keepdims=True) + 1.0
    dis_ref[...] = 1.0 / jnp.sqrt(s)


_dis_call = pl.pallas_call(
    _dis_body,
    out_shape=jax.ShapeDtypeStruct((1, NPAD), jnp.float32),
    grid=(NPAD // 256,),
    in_specs=[pl.BlockSpec((W, 256), lambda j: (0, j))],
    out_specs=pl.BlockSpec((1, 256), lambda j: (0, j)),
)


def _scale1_body(dis_ref, x_ref, o_ref):
    o_ref[...] = dis_ref[...] * x_ref[...]


_scale1_call = pl.pallas_call(
    _scale1_body,
    out_shape=jax.ShapeDtypeStruct((NPAD, D), jnp.float32),
    grid=(NPAD // 256,),
    in_specs=[
        pl.BlockSpec((256, 1), lambda i: (i, 0)),
        pl.BlockSpec((256, D), lambda i: (i, 0)),
    ],
    out_specs=pl.BlockSpec((256, D), lambda i: (i, 0)),
)


def _scale2_body(dis_ref, p0_ref, p1_ref, g_ref, o_ref):
    d = dis_ref[...]
    o_ref[...] = (d * d) * (p0_ref[...] + p1_ref[...] + g_ref[...])


_scale2_call = pl.pallas_call(
    _scale2_body,
    out_shape=jax.ShapeDtypeStruct((NPAD, D), jnp.float32),
    grid=(NPAD // 256,),
    in_specs=[
        pl.BlockSpec((256, 1), lambda i: (i, 0)),
        pl.BlockSpec((256, D), lambda i: (i, 0)),
        pl.BlockSpec((256, D), lambda i: (i, 0)),
        pl.BlockSpec((256, D), lambda i: (i, 0)),
    ],
    out_specs=pl.BlockSpec((256, D), lambda i: (i, 0)),
)


def _final_body(dis_ref, p0_ref, p1_ref, g_ref, o_ref):
    i = pl.program_id(0)

    @pl.when(i == 0)
    def _():
        o_ref[...] = jnp.zeros((1, 1), jnp.float32)

    h = dis_ref[...] * (p0_ref[...] + p1_ref[...] + g_ref[...])
    o_ref[...] = o_ref[...] + jnp.sum(h * h)

    @pl.when(i == pl.num_programs(0) - 1)
    def _():
        o_ref[...] = jnp.sqrt(o_ref[...])


_final_call = pl.pallas_call(
    _final_body,
    out_shape=jax.ShapeDtypeStruct((1, 1), jnp.float32),
    grid=(NPAD // 256,),
    in_specs=[
        pl.BlockSpec((256, 1), lambda i: (i, 0)),
        pl.BlockSpec((256, D), lambda i: (i, 0)),
        pl.BlockSpec((256, D), lambda i: (i, 0)),
        pl.BlockSpec((256, D), lambda i: (i, 0)),
    ],
    out_specs=pl.BlockSpec((1, 1), lambda i: (0, 0)),
)


def kernel(x, adj, pool):
    row = adj[0]
    col = adj[1]
    x_pad = jnp.zeros((NPAD, D), jnp.float32).at[:N].set(x)
    zeros_rows = jnp.zeros((RS, D), jnp.float32)

    degparts = _deg_kernel(col)
    dis_row = _dis_call(degparts)
    dis_col = dis_row.reshape(NPAD, 1)

    g = _scale1_call(dis_col, x_pad)
    for r in range(3):
        p0, p1 = _prop_kernel(g, row, col, zeros_rows)
        if r < 2:
            g = _scale2_call(dis_col, p0, p1, g)
        else:
            out = _final_call(dis_col, p0, p1, g)
    return out.reshape(())


# speedup vs baseline: 19.7565x; 1.2550x over previous
import functools

import jax
import jax.numpy as jnp
from jax import lax
from jax.experimental import pallas as pl
from jax.experimental.pallas import tpu as pltpu
from jax.experimental.pallas import tpu_sc as plsc

N = 10000
E = 320000
D = 128
NPAD = 10240
NC = 2
NS = 16
W = NC * NS
EW = E // W
C = 128
EPW = 10240
EPAD = W * EPW
NCHUNK = EPW // C
RS = NPAD // NS
DEGC = 2000
PACK = 16384

_mesh = plsc.VectorSubcoreMesh(core_axis_name="c", subcore_axis_name="s")


@functools.partial(
    pl.kernel,
    out_type=jax.ShapeDtypeStruct((W, NPAD), jnp.float32),
    mesh=_mesh,
    compiler_params=pltpu.CompilerParams(needs_layout_passes=False),
    scratch_types=[
        pltpu.VMEM((NPAD,), jnp.float32),
        pltpu.VMEM((DEGC,), jnp.int32),
    ],
)
def _deg_kernel(col_hbm, degparts_hbm, degbuf, colbuf):
    cid = lax.axis_index("c")
    sid = lax.axis_index("s")
    w = sid * NC + cid

    def zero(i, carry):
        degbuf[pl.ds(i * 16, 16)] = jnp.zeros((16,), jnp.float32)
        return carry

    lax.fori_loop(0, NPAD // 16, zero, None)

    ones = jnp.ones((16,), jnp.float32)

    def chunk(k, carry):
        pltpu.sync_copy(col_hbm.at[pl.ds(w * EW + k * DEGC, DEGC)], colbuf)

        def scat(j, c2):
            idx = colbuf[pl.ds(j * 16, 16)]
            plsc.addupdate_scatter(degbuf, [idx], ones)
            return c2

        lax.fori_loop(0, DEGC // 16, scat, None)
        return carry

    lax.fori_loop(0, EW // DEGC, chunk, None)
    pltpu.sync_copy(degbuf, degparts_hbm.at[w])


@functools.partial(
    pl.kernel,
    out_type=[
        jax.ShapeDtypeStruct((NPAD, D), jnp.float32),
        jax.ShapeDtypeStruct((NPAD, D), jnp.float32),
    ],
    mesh=_mesh,
    compiler_params=pltpu.CompilerParams(needs_layout_passes=False),
    scratch_types=[
        pltpu.VMEM_SHARED((NPAD, D), jnp.float32),
        pltpu.VMEM((C, D), jnp.float32),
        pltpu.VMEM((C, D), jnp.float32),
        pltpu.VMEM((NCHUNK, C), jnp.int32),
        pltpu.VMEM((C,), jnp.int32),
        pltpu.VMEM((C,), jnp.int32),
        pltpu.VMEM((C,), jnp.int32),
        pltpu.VMEM((C,), jnp.int32),
        pltpu.SemaphoreType.DMA,
        pltpu.SemaphoreType.DMA,
        pltpu.SemaphoreType.DMA,
        pltpu.SemaphoreType.DMA,
    ],
)
def _prop_kernel(g_hbm, packed3_hbm, zeros_hbm,
                 p0_hbm, p1_hbm,
                 acc_sh, rows0, rows1, packed_all, ir0, ic0, ir1, ic1,
                 gsem0, gsem1, ssem0, ssem1):
    cid = lax.axis_index("c")
    sid = lax.axis_index("s")
    w = sid * NC + cid
    rbase = sid * RS

    pltpu.sync_copy(zeros_hbm, acc_sh.at[pl.ds(rbase, RS)])
    pltpu.sync_copy(packed3_hbm.at[w], packed_all)
    plsc.subcore_barrier()

    def unpack(k, irb, icb):
        def lane(j, carry):
            p16 = packed_all[k, pl.ds(j * 16, 16)]
            irb[pl.ds(j * 16, 16)] = lax.shift_right_logical(p16, 14)
            icb[pl.ds(j * 16, 16)] = lax.bitwise_and(p16, PACK - 1)
            return carry

        lax.fori_loop(0, C // 16, lane, None)

    def gfire(irb, buf, sem):
        pltpu.async_copy(g_hbm.at[irb], buf, sem)

    def gwait(irb, buf, sem):
        pltpu.make_async_copy(g_hbm.at[irb], buf, sem).wait()

    def sfire(icb, buf, sem):
        pltpu.async_copy(buf, acc_sh.at[icb], sem, add=True)

    def swait(icb, buf, sem):
        pltpu.make_async_copy(buf, acc_sh.at[icb], sem).wait()

    unpack(0, ir0, ic0)
    gfire(ir0, rows0, gsem0)
    unpack(1, ir1, ic1)
    gfire(ir1, rows1, gsem1)

    def pair(i, carry):
        k0 = 2 * i
        gwait(ir0, rows0, gsem0)
        sfire(ic0, rows0, ssem0)
        gwait(ir1, rows1, gsem1)
        sfire(ic1, rows1, ssem1)
        swait(ic0, rows0, ssem0)
        unpack(k0 + 2, ir0, ic0)
        gfire(ir0, rows0, gsem0)
        swait(ic1, rows1, ssem1)
        unpack(k0 + 3, ir1, ic1)
        gfire(ir1, rows1, gsem1)
        return carry

    lax.fori_loop(0, NCHUNK // 2 - 1, pair, None)
    gwait(ir0, rows0, gsem0)
    sfire(ic0, rows0, ssem0)
    gwait(ir1, rows1, gsem1)
    sfire(ic1, rows1, ssem1)
    swait(ic0, rows0, ssem0)
    swait(ic1, rows1, ssem1)
    plsc.subcore_barrier()

    @pl.when(cid == 0)
    def _():
        pltpu.sync_copy(acc_sh.at[pl.ds(rbase, RS)], p0_hbm.at[pl.ds(rbase, RS)])

    @pl.when(cid == 1)
    def _():
        pltpu.sync_copy(acc_sh.at[pl.ds(rbase, RS)], p1_hbm.at[pl.ds(rbase, RS)])


def _dis_body(dp_ref, dis_ref):
    s = jnp.sum(dp_ref[...], axis=0, keepdims=True) + 1.0
    dis_ref[...] = 1.0 / jnp.sqrt(s)


_dis_call = pl.pallas_call(
    _dis_body,
    out_shape=jax.ShapeDtypeStruct((1, NPAD), jnp.float32),
    grid=(NPAD // 256,),
    in_specs=[pl.BlockSpec((W, 256), lambda j: (0, j))],
    out_specs=pl.BlockSpec((1, 256), lambda j: (0, j)),
)


def _scale1_body(dis_ref, x_ref, o_ref):
    o_ref[...] = dis_ref[...] * x_ref[...]


_scale1_call = pl.pallas_call(
    _scale1_body,
    out_shape=jax.ShapeDtypeStruct((NPAD, D), jnp.float32),
    grid=(NPAD // 256,),
    in_specs=[
        pl.BlockSpec((256, 1), lambda i: (i, 0)),
        pl.BlockSpec((256, D), lambda i: (i, 0)),
    ],
    out_specs=pl.BlockSpec((256, D), lambda i: (i, 0)),
)


def _scale2_body(dis_ref, p0_ref, p1_ref, g_ref, o_ref):
    d = dis_ref[...]
    o_ref[...] = (d * d) * (p0_ref[...] + p1_ref[...] + g_ref[...])


_scale2_call = pl.pallas_call(
    _scale2_body,
    out_shape=jax.ShapeDtypeStruct((NPAD, D), jnp.float32),
    grid=(NPAD // 256,),
    in_specs=[
        pl.BlockSpec((256, 1), lambda i: (i, 0)),
        pl.BlockSpec((256, D), lambda i: (i, 0)),
        pl.BlockSpec((256, D), lambda i: (i, 0)),
        pl.BlockSpec((256, D), lambda i: (i, 0)),
    ],
    out_specs=pl.BlockSpec((256, D), lambda i: (i, 0)),
)


def _final_body(dis_ref, p0_ref, p1_ref, g_ref, o_ref):
    i = pl.program_id(0)

    @pl.when(i == 0)
    def _():
        o_ref[...] = jnp.zeros((1, 1), jnp.float32)

    h = dis_ref[...] * (p0_ref[...] + p1_ref[...] + g_ref[...])
    o_ref[...] = o_ref[...] + jnp.sum(h * h)

    @pl.when(i == pl.num_programs(0) - 1)
    def _():
        o_ref[...] = jnp.sqrt(o_ref[...])


_final_call = pl.pallas_call(
    _final_body,
    out_shape=jax.ShapeDtypeStruct((1, 1), jnp.float32),
    grid=(NPAD // 256,),
    in_specs=[
        pl.BlockSpec((256, 1), lambda i: (i, 0)),
        pl.BlockSpec((256, D), lambda i: (i, 0)),
        pl.BlockSpec((256, D), lambda i: (i, 0)),
        pl.BlockSpec((256, D), lambda i: (i, 0)),
    ],
    out_specs=pl.BlockSpec((1, 1), lambda i: (0, 0)),
)


def kernel(x, adj, pool):
    row = adj[0]
    col = adj[1]
    npad_ids = N + (jnp.arange(EPAD - E, dtype=jnp.int32) % (NPAD - N))
    rowp = jnp.concatenate([row, npad_ids])
    colp = jnp.concatenate([col, npad_ids])
    packed3 = (rowp * PACK + colp).reshape(W, NCHUNK, C)
    x_pad = jnp.zeros((NPAD, D), jnp.float32).at[:N].set(x)
    zeros_rows = jnp.zeros((RS, D), jnp.float32)

    degparts = _deg_kernel(col)
    dis_row = _dis_call(degparts)
    dis_col = dis_row.reshape(NPAD, 1)

    g = _scale1_call(dis_col, x_pad)
    for r in range(3):
        p0, p1 = _prop_kernel(g, packed3, zeros_rows)
        if r < 2:
            g = _scale2_call(dis_col, p0, p1, g)
        else:
            out = _final_call(dis_col, p0, p1, g)
    return out.reshape(())


# speedup vs baseline: 22.2255x; 1.1250x over previous
import functools

import jax
import jax.numpy as jnp
from jax import lax
from jax.experimental import pallas as pl
from jax.experimental.pallas import tpu as pltpu
from jax.experimental.pallas import tpu_sc as plsc

N = 10000
E = 320000
D = 128
NPAD = 10240
NC = 2
NS = 16
W = NC * NS
EW = E // W
C = 80
EPW = 10240
EPAD = W * EPW
NCHUNK = EPW // C
RS = NPAD // NS
DEGC = 2000
PACK = 16384

_mesh = plsc.VectorSubcoreMesh(core_axis_name="c", subcore_axis_name="s")


@functools.partial(
    pl.kernel,
    out_type=jax.ShapeDtypeStruct((W, NPAD), jnp.float32),
    mesh=_mesh,
    compiler_params=pltpu.CompilerParams(needs_layout_passes=False),
    scratch_types=[
        pltpu.VMEM((NPAD,), jnp.float32),
        pltpu.VMEM((DEGC,), jnp.int32),
    ],
)
def _deg_kernel(col_hbm, degparts_hbm, degbuf, colbuf):
    cid = lax.axis_index("c")
    sid = lax.axis_index("s")
    w = sid * NC + cid

    def zero(i, carry):
        degbuf[pl.ds(i * 16, 16)] = jnp.zeros((16,), jnp.float32)
        return carry

    lax.fori_loop(0, NPAD // 16, zero, None)

    ones = jnp.ones((16,), jnp.float32)

    def chunk(k, carry):
        pltpu.sync_copy(col_hbm.at[pl.ds(w * EW + k * DEGC, DEGC)], colbuf)

        def scat(j, c2):
            idx = colbuf[pl.ds(j * 16, 16)]
            plsc.addupdate_scatter(degbuf, [idx], ones)
            return c2

        lax.fori_loop(0, DEGC // 16, scat, None)
        return carry

    lax.fori_loop(0, EW // DEGC, chunk, None)
    pltpu.sync_copy(degbuf, degparts_hbm.at[w])


@functools.partial(
    pl.kernel,
    out_type=[
        jax.ShapeDtypeStruct((NPAD, D), jnp.float32),
        jax.ShapeDtypeStruct((NPAD, D), jnp.float32),
    ],
    mesh=_mesh,
    compiler_params=pltpu.CompilerParams(needs_layout_passes=False),
    scratch_types=[
        pltpu.VMEM_SHARED((NPAD, D), jnp.float32),
        pltpu.VMEM((C, D), jnp.float32),
        pltpu.VMEM((C, D), jnp.float32),
        pltpu.VMEM((C, D), jnp.float32),
        pltpu.VMEM((EPW,), jnp.int32),
        pltpu.VMEM((C,), jnp.int32),
        pltpu.VMEM((C,), jnp.int32),
        pltpu.VMEM((C,), jnp.int32),
        pltpu.VMEM((C,), jnp.int32),
        pltpu.VMEM((C,), jnp.int32),
        pltpu.VMEM((C,), jnp.int32),
        pltpu.SemaphoreType.DMA,
        pltpu.SemaphoreType.DMA,
        pltpu.SemaphoreType.DMA,
        pltpu.SemaphoreType.DMA,
        pltpu.SemaphoreType.DMA,
        pltpu.SemaphoreType.DMA,
    ],
)
def _prop_kernel(g_hbm, packed3_hbm, zeros_hbm,
                 p0_hbm, p1_hbm,
                 acc_sh, rows0, rows1, rows2, packed_all,
                 ir0, ic0, ir1, ic1, ir2, ic2,
                 gsem0, gsem1, gsem2, ssem0, ssem1, ssem2):
    cid = lax.axis_index("c")
    sid = lax.axis_index("s")
    w = sid * NC + cid
    rbase = sid * RS

    pltpu.sync_copy(zeros_hbm, acc_sh.at[pl.ds(rbase, RS)])
    pltpu.sync_copy(packed3_hbm.at[w], packed_all)
    plsc.subcore_barrier()

    rows = (rows0, rows1, rows2)
    irs = (ir0, ir1, ir2)
    ics = (ic0, ic1, ic2)
    gsems = (gsem0, gsem1, gsem2)
    ssems = (ssem0, ssem1, ssem2)

    def unpack(k, irb, icb):
        def lane(j, carry):
            p16 = packed_all[pl.ds(k * C + j * 16, 16)]
            irb[pl.ds(j * 16, 16)] = lax.shift_right_logical(p16, 14)
            icb[pl.ds(j * 16, 16)] = lax.bitwise_and(p16, PACK - 1)
            return carry

        lax.fori_loop(0, C // 16, lane, None)

    def gfire(b):
        pltpu.async_copy(g_hbm.at[irs[b]], rows[b], gsems[b])

    def gwait(b):
        pltpu.make_async_copy(g_hbm.at[irs[b]], rows[b], gsems[b]).wait()

    def sfire(b):
        pltpu.async_copy(rows[b], acc_sh.at[ics[b]], ssems[b], add=True)

    def swait(b):
        pltpu.make_async_copy(rows[b], acc_sh.at[ics[b]], ssems[b]).wait()

    for b in range(3):
        unpack(b, irs[b], ics[b])
        gfire(b)

    NTRIP = NCHUNK // 3

    def triple(i, carry):
        k = 3 * i
        for b in range(3):
            gwait(b)
            sfire(b)
        for b in range(3):
            swait(b)
            unpack(jnp.minimum(k + 3 + b, NCHUNK - 1), irs[b], ics[b])
            gfire(b)
        return carry

    lax.fori_loop(0, NTRIP, triple, None)
    gwait(0)
    sfire(0)
    gwait(1)
    sfire(1)
    gwait(2)
    swait(0)
    swait(1)
    plsc.subcore_barrier()

    @pl.when(cid == 0)
    def _():
        pltpu.sync_copy(acc_sh.at[pl.ds(rbase, RS)], p0_hbm.at[pl.ds(rbase, RS)])

    @pl.when(cid == 1)
    def _():
        pltpu.sync_copy(acc_sh.at[pl.ds(rbase, RS)], p1_hbm.at[pl.ds(rbase, RS)])


def _dis_body(dp_ref, dis_ref):
    s = jnp.sum(dp_ref[...], axis=0, keepdims=True) + 1.0
    dis_ref[...] = 1.0 / jnp.sqrt(s)


_dis_call = pl.pallas_call(
    _dis_body,
    out_shape=jax.ShapeDtypeStruct((1, NPAD), jnp.float32),
    grid=(NPAD // 256,),
    in_specs=[pl.BlockSpec((W, 256), lambda j: (0, j))],
    out_specs=pl.BlockSpec((1, 256), lambda j: (0, j)),
)


def _scale1_body(dis_ref, x_ref, o_ref):
    o_ref[...] = dis_ref[...] * x_ref[...]


_scale1_call = pl.pallas_call(
    _scale1_body,
    out_shape=jax.ShapeDtypeStruct((NPAD, D), jnp.float32),
    grid=(NPAD // 256,),
    in_specs=[
        pl.BlockSpec((256, 1), lambda i: (i, 0)),
        pl.BlockSpec((256, D), lambda i: (i, 0)),
    ],
    out_specs=pl.BlockSpec((256, D), lambda i: (i, 0)),
)


def _scale2_body(dis_ref, p0_ref, p1_ref, g_ref, o_ref):
    d = dis_ref[...]
    o_ref[...] = (d * d) * (p0_ref[...] + p1_ref[...] + g_ref[...])


_scale2_call = pl.pallas_call(
    _scale2_body,
    out_shape=jax.ShapeDtypeStruct((NPAD, D), jnp.float32),
    grid=(NPAD // 256,),
    in_specs=[
        pl.BlockSpec((256, 1), lambda i: (i, 0)),
        pl.BlockSpec((256, D), lambda i: (i, 0)),
        pl.BlockSpec((256, D), lambda i: (i, 0)),
        pl.BlockSpec((256, D), lambda i: (i, 0)),
    ],
    out_specs=pl.BlockSpec((256, D), lambda i: (i, 0)),
)


def _final_body(dis_ref, p0_ref, p1_ref, g_ref, o_ref):
    i = pl.program_id(0)

    @pl.when(i == 0)
    def _():
        o_ref[...] = jnp.zeros((1, 1), jnp.float32)

    h = dis_ref[...] * (p0_ref[...] + p1_ref[...] + g_ref[...])
    o_ref[...] = o_ref[...] + jnp.sum(h * h)

    @pl.when(i == pl.num_programs(0) - 1)
    def _():
        o_ref[...] = jnp.sqrt(o_ref[...])


_final_call = pl.pallas_call(
    _final_body,
    out_shape=jax.ShapeDtypeStruct((1, 1), jnp.float32),
    grid=(NPAD // 256,),
    in_specs=[
        pl.BlockSpec((256, 1), lambda i: (i, 0)),
        pl.BlockSpec((256, D), lambda i: (i, 0)),
        pl.BlockSpec((256, D), lambda i: (i, 0)),
        pl.BlockSpec((256, D), lambda i: (i, 0)),
    ],
    out_specs=pl.BlockSpec((1, 1), lambda i: (0, 0)),
)


def kernel(x, adj, pool):
    row = adj[0]
    col = adj[1]
    npad_ids = N + (jnp.arange(EPAD - E, dtype=jnp.int32) % (NPAD - N))
    rowp = jnp.concatenate([row, npad_ids])
    colp = jnp.concatenate([col, npad_ids])
    packed3 = (rowp * PACK + colp).reshape(W, EPW)
    x_pad = jnp.zeros((NPAD, D), jnp.float32).at[:N].set(x)
    zeros_rows = jnp.zeros((RS, D), jnp.float32)

    degparts = _deg_kernel(col)
    dis_row = _dis_call(degparts)
    dis_col = dis_row.reshape(NPAD, 1)

    g = _scale1_call(dis_col, x_pad)
    for r in range(3):
        p0, p1 = _prop_kernel(g, packed3, zeros_rows)
        if r < 2:
            g = _scale2_call(dis_col, p0, p1, g)
        else:
            out = _final_call(dis_col, p0, p1, g)
    return out.reshape(())


# speedup vs baseline: 23.2941x; 1.0481x over previous
import functools

import jax
import jax.numpy as jnp
from jax import lax
from jax.experimental import pallas as pl
from jax.experimental.pallas import tpu as pltpu
from jax.experimental.pallas import tpu_sc as plsc

N = 10000
E = 320000
D = 128
NPAD = 10240
NC = 2
NS = 16
W = NC * NS
EW = E // W
C = 64
EPW = 10240
EPAD = W * EPW
NCHUNK = EPW // C
RS = NPAD // NS
DEGC = 2000
PACK = 16384

_mesh = plsc.VectorSubcoreMesh(core_axis_name="c", subcore_axis_name="s")


@functools.partial(
    pl.kernel,
    out_type=jax.ShapeDtypeStruct((W, NPAD), jnp.float32),
    mesh=_mesh,
    compiler_params=pltpu.CompilerParams(needs_layout_passes=False),
    scratch_types=[
        pltpu.VMEM((NPAD,), jnp.float32),
        pltpu.VMEM((DEGC,), jnp.int32),
    ],
)
def _deg_kernel(col_hbm, degparts_hbm, degbuf, colbuf):
    cid = lax.axis_index("c")
    sid = lax.axis_index("s")
    w = sid * NC + cid

    def zero(i, carry):
        degbuf[pl.ds(i * 16, 16)] = jnp.zeros((16,), jnp.float32)
        return carry

    lax.fori_loop(0, NPAD // 16, zero, None)

    ones = jnp.ones((16,), jnp.float32)

    def chunk(k, carry):
        pltpu.sync_copy(col_hbm.at[pl.ds(w * EW + k * DEGC, DEGC)], colbuf)

        def scat(j, c2):
            idx = colbuf[pl.ds(j * 16, 16)]
            plsc.addupdate_scatter(degbuf, [idx], ones)
            return c2

        lax.fori_loop(0, DEGC // 16, scat, None)
        return carry

    lax.fori_loop(0, EW // DEGC, chunk, None)
    pltpu.sync_copy(degbuf, degparts_hbm.at[w])


@functools.partial(
    pl.kernel,
    out_type=[
        jax.ShapeDtypeStruct((NPAD, D), jnp.float32),
        jax.ShapeDtypeStruct((NPAD, D), jnp.float32),
    ],
    mesh=_mesh,
    compiler_params=pltpu.CompilerParams(needs_layout_passes=False),
    scratch_types=[
        pltpu.VMEM_SHARED((NPAD, D), jnp.float32),
        pltpu.VMEM((C, D), jnp.float32),
        pltpu.VMEM((C, D), jnp.float32),
        pltpu.VMEM((C, D), jnp.float32),
        pltpu.VMEM((C, D), jnp.float32),
        pltpu.VMEM((EPW,), jnp.int32),
        pltpu.VMEM((C,), jnp.int32),
        pltpu.VMEM((C,), jnp.int32),
        pltpu.VMEM((C,), jnp.int32),
        pltpu.VMEM((C,), jnp.int32),
        pltpu.VMEM((C,), jnp.int32),
        pltpu.VMEM((C,), jnp.int32),
        pltpu.VMEM((C,), jnp.int32),
        pltpu.VMEM((C,), jnp.int32),
        pltpu.SemaphoreType.DMA,
        pltpu.SemaphoreType.DMA,
        pltpu.SemaphoreType.DMA,
        pltpu.SemaphoreType.DMA,
        pltpu.SemaphoreType.DMA,
        pltpu.SemaphoreType.DMA,
        pltpu.SemaphoreType.DMA,
        pltpu.SemaphoreType.DMA,
    ],
)
def _prop_kernel(g_hbm, packed3_hbm, zeros_hbm,
                 p0_hbm, p1_hbm,
                 acc_sh, rows0, rows1, rows2, rows3, packed_all,
                 ir0, ic0, ir1, ic1, ir2, ic2, ir3, ic3,
                 gsem0, gsem1, gsem2, gsem3, ssem0, ssem1, ssem2, ssem3):
    cid = lax.axis_index("c")
    sid = lax.axis_index("s")
    w = sid * NC + cid
    rbase = sid * RS

    pltpu.sync_copy(zeros_hbm, acc_sh.at[pl.ds(rbase, RS)])
    pltpu.sync_copy(packed3_hbm.at[w], packed_all)
    plsc.subcore_barrier()

    rows = (rows0, rows1, rows2, rows3)
    irs = (ir0, ir1, ir2, ir3)
    ics = (ic0, ic1, ic2, ic3)
    gsems = (gsem0, gsem1, gsem2, gsem3)
    ssems = (ssem0, ssem1, ssem2, ssem3)

    def unpack(k, irb, icb):
        def lane(j, carry):
            p16 = packed_all[pl.ds(k * C + j * 16, 16)]
            irb[pl.ds(j * 16, 16)] = lax.shift_right_logical(p16, 14)
            icb[pl.ds(j * 16, 16)] = lax.bitwise_and(p16, PACK - 1)
            return carry

        lax.fori_loop(0, C // 16, lane, None)

    def gfire(b):
        pltpu.async_copy(g_hbm.at[irs[b]], rows[b], gsems[b])

    def gwait(b):
        pltpu.make_async_copy(g_hbm.at[irs[b]], rows[b], gsems[b]).wait()

    def sfire(b):
        pltpu.async_copy(rows[b], acc_sh.at[ics[b]], ssems[b], add=True)

    def swait(b):
        pltpu.make_async_copy(rows[b], acc_sh.at[ics[b]], ssems[b]).wait()

    NB = 4
    for b in range(NB):
        unpack(b, irs[b], ics[b])
        gfire(b)

    NGRP = NCHUNK // NB - 1

    def group(i, carry):
        k = NB * i
        for b in range(NB):
            gwait(b)
            sfire(b)
        for b in range(NB):
            swait(b)
            unpack(k + NB + b, irs[b], ics[b])
            gfire(b)
        return carry

    lax.fori_loop(0, NGRP, group, None)
    for b in range(NB):
        gwait(b)
        sfire(b)
    for b in range(NB):
        swait(b)
    plsc.subcore_barrier()

    @pl.when(cid == 0)
    def _():
        pltpu.sync_copy(acc_sh.at[pl.ds(rbase, RS)], p0_hbm.at[pl.ds(rbase, RS)])

    @pl.when(cid == 1)
    def _():
        pltpu.sync_copy(acc_sh.at[pl.ds(rbase, RS)], p1_hbm.at[pl.ds(rbase, RS)])


def _dis_body(dp_ref, dis_ref):
    s = jnp.sum(dp_ref[...], axis=0, keepdims=True) + 1.0
    dis_ref[...] = 1.0 / jnp.sqrt(s)


_dis_call = pl.pallas_call(
    _dis_body,
    out_shape=jax.ShapeDtypeStruct((1, NPAD), jnp.float32),
    grid=(NPAD // 256,),
    in_specs=[pl.BlockSpec((W, 256), lambda j: (0, j))],
    out_specs=pl.BlockSpec((1, 256), lambda j: (0, j)),
)


def _scale1_body(dis_ref, x_ref, o_ref):
    o_ref[...] = dis_ref[...] * x_ref[...]


_scale1_call = pl.pallas_call(
    _scale1_body,
    out_shape=jax.ShapeDtypeStruct((NPAD, D), jnp.float32),
    grid=(NPAD // 256,),
    in_specs=[
        pl.BlockSpec((256, 1), lambda i: (i, 0)),
        pl.BlockSpec((256, D), lambda i: (i, 0)),
    ],
    out_specs=pl.BlockSpec((256, D), lambda i: (i, 0)),
)


def _scale2_body(dis_ref, p0_ref, p1_ref, g_ref, o_ref):
    d = dis_ref[...]
    o_ref[...] = (d * d) * (p0_ref[...] + p1_ref[...] + g_ref[...])


_scale2_call = pl.pallas_call(
    _scale2_body,
    out_shape=jax.ShapeDtypeStruct((NPAD, D), jnp.float32),
    grid=(NPAD // 256,),
    in_specs=[
        pl.BlockSpec((256, 1), lambda i: (i, 0)),
        pl.BlockSpec((256, D), lambda i: (i, 0)),
        pl.BlockSpec((256, D), lambda i: (i, 0)),
        pl.BlockSpec((256, D), lambda i: (i, 0)),
    ],
    out_specs=pl.BlockSpec((256, D), lambda i: (i, 0)),
)


def _final_body(dis_ref, p0_ref, p1_ref, g_ref, o_ref):
    i = pl.program_id(0)

    @pl.when(i == 0)
    def _():
        o_ref[...] = jnp.zeros((1, 1), jnp.float32)

    h = dis_ref[...] * (p0_ref[...] + p1_ref[...] + g_ref[...])
    o_ref[...] = o_ref[...] + jnp.sum(h * h)

    @pl.when(i == pl.num_programs(0) - 1)
    def _():
        o_ref[...] = jnp.sqrt(o_ref[...])


_final_call = pl.pallas_call(
    _final_body,
    out_shape=jax.ShapeDtypeStruct((1, 1), jnp.float32),
    grid=(NPAD // 256,),
    in_specs=[
        pl.BlockSpec((256, 1), lambda i: (i, 0)),
        pl.BlockSpec((256, D), lambda i: (i, 0)),
        pl.BlockSpec((256, D), lambda i: (i, 0)),
        pl.BlockSpec((256, D), lambda i: (i, 0)),
    ],
    out_specs=pl.BlockSpec((1, 1), lambda i: (0, 0)),
)


def kernel(x, adj, pool):
    row = adj[0]
    col = adj[1]
    npad_ids = N + (jnp.arange(EPAD - E, dtype=jnp.int32) % (NPAD - N))
    rowp = jnp.concatenate([row, npad_ids])
    colp = jnp.concatenate([col, npad_ids])
    packed3 = (rowp * PACK + colp).reshape(W, EPW)
    x_pad = jnp.zeros((NPAD, D), jnp.float32).at[:N].set(x)
    zeros_rows = jnp.zeros((RS, D), jnp.float32)

    degparts = _deg_kernel(col)
    dis_row = _dis_call(degparts)
    dis_col = dis_row.reshape(NPAD, 1)

    g = _scale1_call(dis_col, x_pad)
    for r in range(3):
        p0, p1 = _prop_kernel(g, packed3, zeros_rows)
        if r < 2:
            g = _scale2_call(dis_col, p0, p1, g)
        else:
            out = _final_call(dis_col, p0, p1, g)
    return out.reshape(())


# speedup vs baseline: 23.8888x; 1.0255x over previous
import functools

import jax
import jax.numpy as jnp
from jax import lax
from jax.experimental import pallas as pl
from jax.experimental.pallas import tpu as pltpu
from jax.experimental.pallas import tpu_sc as plsc

N = 10000
E = 320000
D = 128
NPAD = 10240
NC = 2
NS = 16
W = NC * NS
EW = E // W
C = 64
EPW = 10240
EPAD = W * EPW
NCHUNK = EPW // C
RS = NPAD // NS
DEGC = 2000
PACK = 16384

_mesh = plsc.VectorSubcoreMesh(core_axis_name="c", subcore_axis_name="s")


@functools.partial(
    pl.kernel,
    out_type=jax.ShapeDtypeStruct((W, NPAD), jnp.float32),
    mesh=_mesh,
    compiler_params=pltpu.CompilerParams(needs_layout_passes=False),
    scratch_types=[
        pltpu.VMEM((NPAD,), jnp.float32),
        pltpu.VMEM((DEGC,), jnp.int32),
    ],
)
def _deg_kernel(col_hbm, degparts_hbm, degbuf, colbuf):
    cid = lax.axis_index("c")
    sid = lax.axis_index("s")
    w = sid * NC + cid

    def zero(i, carry):
        degbuf[pl.ds(i * 16, 16)] = jnp.zeros((16,), jnp.float32)
        return carry

    lax.fori_loop(0, NPAD // 16, zero, None)

    ones = jnp.ones((16,), jnp.float32)

    def chunk(k, carry):
        pltpu.sync_copy(col_hbm.at[pl.ds(w * EW + k * DEGC, DEGC)], colbuf)

        def scat(j, c2):
            idx = colbuf[pl.ds(j * 16, 16)]
            plsc.addupdate_scatter(degbuf, [idx], ones)
            return c2

        lax.fori_loop(0, DEGC // 16, scat, None)
        return carry

    lax.fori_loop(0, EW // DEGC, chunk, None)
    pltpu.sync_copy(degbuf, degparts_hbm.at[w])


@functools.partial(
    pl.kernel,
    out_type=[
        jax.ShapeDtypeStruct((NPAD, D), jnp.float32),
        jax.ShapeDtypeStruct((NPAD, D), jnp.float32),
    ],
    mesh=_mesh,
    compiler_params=pltpu.CompilerParams(needs_layout_passes=False),
    scratch_types=[
        pltpu.VMEM_SHARED((NPAD, D), jnp.float32),
        [pltpu.VMEM((C, D), jnp.float32) for _ in range(5)],
        [pltpu.VMEM((C,), jnp.int32) for _ in range(5)],
        [pltpu.VMEM((C,), jnp.int32) for _ in range(5)],
        [pltpu.VMEM((C,), jnp.int32) for _ in range(5)],
        [pltpu.SemaphoreType.DMA for _ in range(5)],
        [pltpu.SemaphoreType.DMA for _ in range(5)],
        [pltpu.SemaphoreType.DMA for _ in range(5)],
    ],
)
def _prop_kernel(g_hbm, packed3_hbm, zeros_hbm,
                 p0_hbm, p1_hbm,
                 acc_sh, rows, pbufs, irs, ics, psems, gsems, ssems):
    cid = lax.axis_index("c")
    sid = lax.axis_index("s")
    w = sid * NC + cid
    rbase = sid * RS

    pltpu.sync_copy(zeros_hbm, acc_sh.at[pl.ds(rbase, RS)])
    plsc.subcore_barrier()

    NB = 5

    def pfire(k, b):
        pltpu.async_copy(packed3_hbm.at[w, pl.ds(k * C, C)], pbufs[b], psems[b])

    def pwait(k, b):
        pltpu.make_async_copy(
            packed3_hbm.at[w, pl.ds(k * C, C)], pbufs[b], psems[b]).wait()

    def unpack(b):
        def lane(j, carry):
            p16 = pbufs[b][pl.ds(j * 16, 16)]
            irs[b][pl.ds(j * 16, 16)] = lax.shift_right_logical(p16, 14)
            ics[b][pl.ds(j * 16, 16)] = lax.bitwise_and(p16, PACK - 1)
            return carry

        lax.fori_loop(0, C // 16, lane, None)

    def gfire(b):
        pltpu.async_copy(g_hbm.at[irs[b]], rows[b], gsems[b])

    def gwait(b):
        pltpu.make_async_copy(g_hbm.at[irs[b]], rows[b], gsems[b]).wait()

    def sfire(b):
        pltpu.async_copy(rows[b], acc_sh.at[ics[b]], ssems[b], add=True)

    def swait(b):
        pltpu.make_async_copy(rows[b], acc_sh.at[ics[b]], ssems[b]).wait()

    for b in range(NB):
        pfire(b, b)
    for b in range(NB):
        pwait(b, b)
        unpack(b)
        pfire(b + NB, b)
        gfire(b)

    NGRP = NCHUNK // NB

    def group(i, carry):
        k = NB * i
        for b in range(NB):
            gwait(b)
            sfire(b)
        for b in range(NB):
            swait(b)
            pwait(k + NB + b, b)
            unpack(b)
            pfire(jnp.minimum(k + 2 * NB + b, NCHUNK - 1), b)
            gfire(b)
        return carry

    lax.fori_loop(0, NGRP - 1, group, None)
    for b in range(NB):
        gwait(b)
        sfire(b)
    for b in range(NB):
        swait(b)
        pwait(NCHUNK - 1, b)
    plsc.subcore_barrier()

    @pl.when(cid == 0)
    def _():
        pltpu.sync_copy(acc_sh.at[pl.ds(rbase, RS)], p0_hbm.at[pl.ds(rbase, RS)])

    @pl.when(cid == 1)
    def _():
        pltpu.sync_copy(acc_sh.at[pl.ds(rbase, RS)], p1_hbm.at[pl.ds(rbase, RS)])


def _dis_body(dp_ref, dis_ref):
    s = jnp.sum(dp_ref[...], axis=0, keepdims=True) + 1.0
    dis_ref[...] = 1.0 / jnp.sqrt(s)


_dis_call = pl.pallas_call(
    _dis_body,
    out_shape=jax.ShapeDtypeStruct((1, NPAD), jnp.float32),
    grid=(NPAD // 256,),
    in_specs=[pl.BlockSpec((W, 256), lambda j: (0, j))],
    out_specs=pl.BlockSpec((1, 256), lambda j: (0, j)),
)


def _scale1_body(dis_ref, x_ref, o_ref):
    o_ref[...] = dis_ref[...] * x_ref[...]


_scale1_call = pl.pallas_call(
    _scale1_body,
    out_shape=jax.ShapeDtypeStruct((NPAD, D), jnp.float32),
    grid=(NPAD // 256,),
    in_specs=[
        pl.BlockSpec((256, 1), lambda i: (i, 0)),
        pl.BlockSpec((256, D), lambda i: (i, 0)),
    ],
    out_specs=pl.BlockSpec((256, D), lambda i: (i, 0)),
)


def _scale2_body(dis_ref, p0_ref, p1_ref, g_ref, o_ref):
    d = dis_ref[...]
    o_ref[...] = (d * d) * (p0_ref[...] + p1_ref[...] + g_ref[...])


_scale2_call = pl.pallas_call(
    _scale2_body,
    out_shape=jax.ShapeDtypeStruct((NPAD, D), jnp.float32),
    grid=(NPAD // 256,),
    in_specs=[
        pl.BlockSpec((256, 1), lambda i: (i, 0)),
        pl.BlockSpec((256, D), lambda i: (i, 0)),
        pl.BlockSpec((256, D), lambda i: (i, 0)),
        pl.BlockSpec((256, D), lambda i: (i, 0)),
    ],
    out_specs=pl.BlockSpec((256, D), lambda i: (i, 0)),
)


def _final_body(dis_ref, p0_ref, p1_ref, g_ref, o_ref):
    i = pl.program_id(0)

    @pl.when(i == 0)
    def _():
        o_ref[...] = jnp.zeros((1, 1), jnp.float32)

    h = dis_ref[...] * (p0_ref[...] + p1_ref[...] + g_ref[...])
    o_ref[...] = o_ref[...] + jnp.sum(h * h)

    @pl.when(i == pl.num_programs(0) - 1)
    def _():
        o_ref[...] = jnp.sqrt(o_ref[...])


_final_call = pl.pallas_call(
    _final_body,
    out_shape=jax.ShapeDtypeStruct((1, 1), jnp.float32),
    grid=(NPAD // 256,),
    in_specs=[
        pl.BlockSpec((256, 1), lambda i: (i, 0)),
        pl.BlockSpec((256, D), lambda i: (i, 0)),
        pl.BlockSpec((256, D), lambda i: (i, 0)),
        pl.BlockSpec((256, D), lambda i: (i, 0)),
    ],
    out_specs=pl.BlockSpec((1, 1), lambda i: (0, 0)),
)


def kernel(x, adj, pool):
    row = adj[0]
    col = adj[1]
    npad_ids = N + (jnp.arange(EPAD - E, dtype=jnp.int32) % (NPAD - N))
    rowp = jnp.concatenate([row, npad_ids])
    colp = jnp.concatenate([col, npad_ids])
    packed3 = (rowp * PACK + colp).reshape(W, EPW)
    x_pad = jnp.zeros((NPAD, D), jnp.float32).at[:N].set(x)
    zeros_rows = jnp.zeros((RS, D), jnp.float32)

    degparts = _deg_kernel(col)
    dis_row = _dis_call(degparts)
    dis_col = dis_row.reshape(NPAD, 1)

    g = _scale1_call(dis_col, x_pad)
    for r in range(3):
        p0, p1 = _prop_kernel(g, packed3, zeros_rows)
        if r < 2:
            g = _scale2_call(dis_col, p0, p1, g)
        else:
            out = _final_call(dis_col, p0, p1, g)
    return out.reshape(())


# speedup vs baseline: 24.9877x; 1.0460x over previous
import functools

import jax
import jax.numpy as jnp
from jax import lax
from jax.experimental import pallas as pl
from jax.experimental.pallas import tpu as pltpu
from jax.experimental.pallas import tpu_sc as plsc

N = 10000
E = 320000
D = 128
NPAD = 10240
NC = 2
NS = 16
W = NC * NS
EW = E // W
C = 64
EPW = 10240
EPAD = W * EPW
NCHUNK = EPW // C
RS = NPAD // NS
DEGC = 2000
PACK = 16384

_mesh = plsc.VectorSubcoreMesh(core_axis_name="c", subcore_axis_name="s")


@functools.partial(
    pl.kernel,
    out_type=jax.ShapeDtypeStruct((W, NPAD), jnp.float32),
    mesh=_mesh,
    compiler_params=pltpu.CompilerParams(needs_layout_passes=False),
    scratch_types=[
        pltpu.VMEM((NPAD,), jnp.float32),
        pltpu.VMEM((DEGC,), jnp.int32),
    ],
)
def _deg_kernel(col_hbm, degparts_hbm, degbuf, colbuf):
    cid = lax.axis_index("c")
    sid = lax.axis_index("s")
    w = sid * NC + cid

    def zero(i, carry):
        degbuf[pl.ds(i * 16, 16)] = jnp.zeros((16,), jnp.float32)
        return carry

    lax.fori_loop(0, NPAD // 16, zero, None)

    ones = jnp.ones((16,), jnp.float32)

    def chunk(k, carry):
        pltpu.sync_copy(col_hbm.at[pl.ds(w * EW + k * DEGC, DEGC)], colbuf)

        def scat(j, c2):
            idx = colbuf[pl.ds(j * 16, 16)]
            plsc.addupdate_scatter(degbuf, [idx], ones)
            return c2

        lax.fori_loop(0, DEGC // 16, scat, None)
        return carry

    lax.fori_loop(0, EW // DEGC, chunk, None)
    pltpu.sync_copy(degbuf, degparts_hbm.at[w])


@functools.partial(
    pl.kernel,
    out_type=[
        jax.ShapeDtypeStruct((NPAD, D), jnp.float32),
        jax.ShapeDtypeStruct((NPAD, D), jnp.float32),
    ],
    mesh=_mesh,
    compiler_params=pltpu.CompilerParams(needs_layout_passes=False),
    scratch_types=[
        pltpu.VMEM_SHARED((NPAD, D), jnp.float32),
        [pltpu.VMEM((C, D), jnp.float32) for _ in range(5)],
        [pltpu.VMEM((C,), jnp.int32) for _ in range(5)],
        [pltpu.VMEM((C,), jnp.int32) for _ in range(5)],
        [pltpu.VMEM((C,), jnp.int32) for _ in range(5)],
        [pltpu.SemaphoreType.DMA for _ in range(5)],
        [pltpu.SemaphoreType.DMA for _ in range(5)],
        [pltpu.SemaphoreType.DMA for _ in range(5)],
    ],
)
def _prop_kernel(g_hbm, packed3_hbm, zeros_hbm,
                 p0_hbm, p1_hbm,
                 acc_sh, rows, pbufs, irs, ics, psems, gsems, ssems):
    cid = lax.axis_index("c")
    sid = lax.axis_index("s")
    w = sid * NC + cid
    rbase = sid * RS

    pltpu.sync_copy(zeros_hbm, acc_sh.at[pl.ds(rbase, RS)])
    plsc.subcore_barrier()

    NB = 5

    def pfire(k, b):
        pltpu.async_copy(packed3_hbm.at[w, pl.ds(k * C, C)], pbufs[b], psems[b])

    def pwait(k, b):
        pltpu.make_async_copy(
            packed3_hbm.at[w, pl.ds(k * C, C)], pbufs[b], psems[b]).wait()

    def unpack(b):
        def lane(j, carry):
            p16 = pbufs[b][pl.ds(j * 16, 16)]
            irs[b][pl.ds(j * 16, 16)] = lax.shift_right_logical(p16, 14)
            ics[b][pl.ds(j * 16, 16)] = lax.bitwise_and(p16, PACK - 1)
            return carry

        lax.fori_loop(0, C // 16, lane, None)

    def gfire(b):
        pltpu.async_copy(g_hbm.at[irs[b]], rows[b], gsems[b])

    def gwait(b):
        pltpu.make_async_copy(g_hbm.at[irs[b]], rows[b], gsems[b]).wait()

    def sfire(b):
        pltpu.async_copy(rows[b], acc_sh.at[ics[b]], ssems[b], add=True)

    def swait(b):
        pltpu.make_async_copy(rows[b], acc_sh.at[ics[b]], ssems[b]).wait()

    for b in range(NB):
        pfire(b, b)
    for b in range(NB):
        pwait(b, b)
        unpack(b)
        pfire(b + NB, b)
        gfire(b)

    NGRP = NCHUNK // NB

    def group(i, carry):
        k = NB * i
        for b in range(NB):
            gwait(b)
            sfire(b)
        for b in range(NB):
            swait(b)
            pwait(k + NB + b, b)
            unpack(b)
            pfire(jnp.minimum(k + 2 * NB + b, NCHUNK - 1), b)
            gfire(b)
        return carry

    lax.fori_loop(0, NGRP - 1, group, None)
    for b in range(NB):
        gwait(b)
        sfire(b)
    for b in range(NB):
        swait(b)
        pwait(NCHUNK - 1, b)
    plsc.subcore_barrier()

    @pl.when(cid == 0)
    def _():
        pltpu.sync_copy(acc_sh.at[pl.ds(rbase, RS)], p0_hbm.at[pl.ds(rbase, RS)])

    @pl.when(cid == 1)
    def _():
        pltpu.sync_copy(acc_sh.at[pl.ds(rbase, RS)], p1_hbm.at[pl.ds(rbase, RS)])


def _scale1_body(dp_ref, x_ref, dis_ref, o_ref):
    s = jnp.sum(dp_ref[...], axis=0, keepdims=True) + 1.0
    d_row = 1.0 / jnp.sqrt(s)
    d_col = d_row.reshape(256, 1)
    dis_ref[...] = d_col
    o_ref[...] = d_col * x_ref[...]


_scale1_call = pl.pallas_call(
    _scale1_body,
    out_shape=[
        jax.ShapeDtypeStruct((NPAD, 1), jnp.float32),
        jax.ShapeDtypeStruct((NPAD, D), jnp.float32),
    ],
    grid=(NPAD // 256,),
    in_specs=[
        pl.BlockSpec((W, 256), lambda i: (0, i)),
        pl.BlockSpec((256, D), lambda i: (i, 0)),
    ],
    out_specs=[
        pl.BlockSpec((256, 1), lambda i: (i, 0)),
        pl.BlockSpec((256, D), lambda i: (i, 0)),
    ],
)


def _scale2_body(dis_ref, p0_ref, p1_ref, g_ref, o_ref):
    d = dis_ref[...]
    o_ref[...] = (d * d) * (p0_ref[...] + p1_ref[...] + g_ref[...])


_scale2_call = pl.pallas_call(
    _scale2_body,
    out_shape=jax.ShapeDtypeStruct((NPAD, D), jnp.float32),
    grid=(NPAD // 256,),
    in_specs=[
        pl.BlockSpec((256, 1), lambda i: (i, 0)),
        pl.BlockSpec((256, D), lambda i: (i, 0)),
        pl.BlockSpec((256, D), lambda i: (i, 0)),
        pl.BlockSpec((256, D), lambda i: (i, 0)),
    ],
    out_specs=pl.BlockSpec((256, D), lambda i: (i, 0)),
)


def _final_body(dis_ref, p0_ref, p1_ref, g_ref, o_ref):
    i = pl.program_id(0)

    @pl.when(i == 0)
    def _():
        o_ref[...] = jnp.zeros((1, 1), jnp.float32)

    h = dis_ref[...] * (p0_ref[...] + p1_ref[...] + g_ref[...])
    o_ref[...] = o_ref[...] + jnp.sum(h * h)

    @pl.when(i == pl.num_programs(0) - 1)
    def _():
        o_ref[...] = jnp.sqrt(o_ref[...])


_final_call = pl.pallas_call(
    _final_body,
    out_shape=jax.ShapeDtypeStruct((1, 1), jnp.float32),
    grid=(NPAD // 256,),
    in_specs=[
        pl.BlockSpec((256, 1), lambda i: (i, 0)),
        pl.BlockSpec((256, D), lambda i: (i, 0)),
        pl.BlockSpec((256, D), lambda i: (i, 0)),
        pl.BlockSpec((256, D), lambda i: (i, 0)),
    ],
    out_specs=pl.BlockSpec((1, 1), lambda i: (0, 0)),
)


def kernel(x, adj, pool):
    row = adj[0]
    col = adj[1]
    npad_ids = N + (jnp.arange(EPAD - E, dtype=jnp.int32) % (NPAD - N))
    rowp = jnp.concatenate([row, npad_ids])
    colp = jnp.concatenate([col, npad_ids])
    packed3 = (rowp * PACK + colp).reshape(W, EPW)
    x_pad = jnp.zeros((NPAD, D), jnp.float32).at[:N].set(x)
    zeros_rows = jnp.zeros((RS, D), jnp.float32)

    degparts = _deg_kernel(col)
    dis_col, g = _scale1_call(degparts, x_pad)
    for r in range(3):
        p0, p1 = _prop_kernel(g, packed3, zeros_rows)
        if r < 2:
            g = _scale2_call(dis_col, p0, p1, g)
        else:
            out = _final_call(dis_col, p0, p1, g)
    return out.reshape(())


# speedup vs baseline: 25.0669x; 1.0032x over previous
import functools

import jax
import jax.numpy as jnp
from jax import lax
from jax.experimental import pallas as pl
from jax.experimental.pallas import tpu as pltpu
from jax.experimental.pallas import tpu_sc as plsc

N = 10000
E = 320000
D = 128
NPAD = 10240
NC = 2
NS = 16
W = NC * NS
EW = E // W
C = 64
EPW = 10240
EPAD = W * EPW
NCHUNK = EPW // C
RS = NPAD // NS
DEGC = 2000
PACK = 16384

_mesh = plsc.VectorSubcoreMesh(core_axis_name="c", subcore_axis_name="s")


@functools.partial(
    pl.kernel,
    out_type=jax.ShapeDtypeStruct((W, NPAD), jnp.float32),
    mesh=_mesh,
    compiler_params=pltpu.CompilerParams(needs_layout_passes=False),
    scratch_types=[
        pltpu.VMEM((NPAD,), jnp.float32),
        pltpu.VMEM((DEGC,), jnp.int32),
    ],
)
def _deg_kernel(col_hbm, degparts_hbm, degbuf, colbuf):
    cid = lax.axis_index("c")
    sid = lax.axis_index("s")
    w = sid * NC + cid

    def zero(i, carry):
        degbuf[pl.ds(i * 16, 16)] = jnp.zeros((16,), jnp.float32)
        return carry

    lax.fori_loop(0, NPAD // 16, zero, None)

    ones = jnp.ones((16,), jnp.float32)

    def chunk(k, carry):
        pltpu.sync_copy(col_hbm.at[pl.ds(w * EW + k * DEGC, DEGC)], colbuf)

        def scat(j, c2):
            idx = colbuf[pl.ds(j * 16, 16)]
            plsc.addupdate_scatter(degbuf, [idx], ones)
            return c2

        lax.fori_loop(0, DEGC // 16, scat, None)
        return carry

    lax.fori_loop(0, EW // DEGC, chunk, None)
    pltpu.sync_copy(degbuf, degparts_hbm.at[w])


@functools.partial(
    pl.kernel,
    out_type=[
        jax.ShapeDtypeStruct((NPAD, D), jnp.float32),
        jax.ShapeDtypeStruct((NPAD, D), jnp.float32),
    ],
    mesh=_mesh,
    compiler_params=pltpu.CompilerParams(needs_layout_passes=False),
    scratch_types=[
        pltpu.VMEM_SHARED((NPAD, D), jnp.float32),
        [pltpu.VMEM((C, D), jnp.float32) for _ in range(5)],
        [pltpu.VMEM((C,), jnp.int32) for _ in range(5)],
        [pltpu.VMEM((C,), jnp.int32) for _ in range(5)],
        [pltpu.VMEM((C,), jnp.int32) for _ in range(5)],
        [pltpu.SemaphoreType.DMA for _ in range(5)],
        [pltpu.SemaphoreType.DMA for _ in range(5)],
        [pltpu.SemaphoreType.DMA for _ in range(5)],
    ],
)
def _prop_kernel(g_hbm, packed3_hbm, zeros_hbm,
                 p0_hbm, p1_hbm,
                 acc_sh, rows, pbufs, irs, ics, psems, gsems, ssems):
    cid = lax.axis_index("c")
    sid = lax.axis_index("s")
    w = sid * NC + cid
    rbase = sid * RS

    pltpu.sync_copy(zeros_hbm, acc_sh.at[pl.ds(rbase, RS)])
    plsc.subcore_barrier()

    NB = 5

    def pfire(k, b):
        pltpu.async_copy(packed3_hbm.at[w, pl.ds(k * C, C)], pbufs[b], psems[b])

    def pwait(k, b):
        pltpu.make_async_copy(
            packed3_hbm.at[w, pl.ds(k * C, C)], pbufs[b], psems[b]).wait()

    def unpack(b):
        def lane(j, carry):
            p16 = pbufs[b][pl.ds(j * 16, 16)]
            irs[b][pl.ds(j * 16, 16)] = lax.shift_right_logical(p16, 14)
            ics[b][pl.ds(j * 16, 16)] = lax.bitwise_and(p16, PACK - 1)
            return carry

        lax.fori_loop(0, C // 16, lane, None)

    def gfire(b):
        pltpu.async_copy(g_hbm.at[irs[b]], rows[b], gsems[b])

    def gwait(b):
        pltpu.make_async_copy(g_hbm.at[irs[b]], rows[b], gsems[b]).wait()

    def sfire(b):
        pltpu.async_copy(rows[b], acc_sh.at[ics[b]], ssems[b], add=True)

    def swait(b):
        pltpu.make_async_copy(rows[b], acc_sh.at[ics[b]], ssems[b]).wait()

    for b in range(NB):
        pfire(b, b)
    for b in range(NB):
        pwait(b, b)
        unpack(b)
        pfire(b + NB, b)
        gfire(b)

    NGRP = NCHUNK // NB

    def group(i, carry):
        k = NB * i
        for b in range(NB):
            gwait(b)
            sfire(b)
        for b in range(NB):
            swait(b)
            pwait(k + NB + b, b)
            unpack(b)
            pfire(jnp.minimum(k + 2 * NB + b, NCHUNK - 1), b)
            gfire(b)
        return carry

    lax.fori_loop(0, NGRP - 1, group, None)
    for b in range(NB):
        gwait(b)
        sfire(b)
    for b in range(NB):
        swait(b)
        pwait(NCHUNK - 1, b)
    plsc.subcore_barrier()

    @pl.when(cid == 0)
    def _():
        pltpu.sync_copy(acc_sh.at[pl.ds(rbase, RS)], p0_hbm.at[pl.ds(rbase, RS)])

    @pl.when(cid == 1)
    def _():
        pltpu.sync_copy(acc_sh.at[pl.ds(rbase, RS)], p1_hbm.at[pl.ds(rbase, RS)])


def _scale1_body(dp_ref, x_ref, dis_ref, o_ref):
    i = pl.program_id(0)
    s = jnp.sum(dp_ref[...], axis=0, keepdims=True) + 1.0
    d_row = 1.0 / jnp.sqrt(s)
    d_col = d_row.reshape(256, 1)
    dis_ref[...] = d_col
    rid = i * 256 + lax.broadcasted_iota(jnp.int32, (256, 1), 0)
    o_ref[...] = jnp.where(rid < N, d_col * x_ref[...], 0.0)


_scale1_call = pl.pallas_call(
    _scale1_body,
    out_shape=[
        jax.ShapeDtypeStruct((NPAD, 1), jnp.float32),
        jax.ShapeDtypeStruct((NPAD, D), jnp.float32),
    ],
    grid=(NPAD // 256,),
    in_specs=[
        pl.BlockSpec((W, 256), lambda i: (0, i)),
        pl.BlockSpec((256, D), lambda i: (i, 0)),
    ],
    out_specs=[
        pl.BlockSpec((256, 1), lambda i: (i, 0)),
        pl.BlockSpec((256, D), lambda i: (i, 0)),
    ],
)


def _scale2_body(dis_ref, p0_ref, p1_ref, g_ref, o_ref):
    d = dis_ref[...]
    o_ref[...] = (d * d) * (p0_ref[...] + p1_ref[...] + g_ref[...])


_scale2_call = pl.pallas_call(
    _scale2_body,
    out_shape=jax.ShapeDtypeStruct((NPAD, D), jnp.float32),
    grid=(NPAD // 256,),
    in_specs=[
        pl.BlockSpec((256, 1), lambda i: (i, 0)),
        pl.BlockSpec((256, D), lambda i: (i, 0)),
        pl.BlockSpec((256, D), lambda i: (i, 0)),
        pl.BlockSpec((256, D), lambda i: (i, 0)),
    ],
    out_specs=pl.BlockSpec((256, D), lambda i: (i, 0)),
)


def _final_body(dis_ref, p0_ref, p1_ref, g_ref, o_ref):
    i = pl.program_id(0)

    @pl.when(i == 0)
    def _():
        o_ref[...] = jnp.zeros((1, 1), jnp.float32)

    h = dis_ref[...] * (p0_ref[...] + p1_ref[...] + g_ref[...])
    o_ref[...] = o_ref[...] + jnp.sum(h * h)

    @pl.when(i == pl.num_programs(0) - 1)
    def _():
        o_ref[...] = jnp.sqrt(o_ref[...])


_final_call = pl.pallas_call(
    _final_body,
    out_shape=jax.ShapeDtypeStruct((1, 1), jnp.float32),
    grid=(NPAD // 256,),
    in_specs=[
        pl.BlockSpec((256, 1), lambda i: (i, 0)),
        pl.BlockSpec((256, D), lambda i: (i, 0)),
        pl.BlockSpec((256, D), lambda i: (i, 0)),
        pl.BlockSpec((256, D), lambda i: (i, 0)),
    ],
    out_specs=pl.BlockSpec((1, 1), lambda i: (0, 0)),
)


def kernel(x, adj, pool):
    row = adj[0]
    col = adj[1]
    npad_ids = N + (jnp.arange(EPAD - E, dtype=jnp.int32) % (NPAD - N))
    rowp = jnp.concatenate([row, npad_ids])
    colp = jnp.concatenate([col, npad_ids])
    packed3 = (rowp * PACK + colp).reshape(W, EPW)
    zeros_rows = jnp.zeros((RS, D), jnp.float32)

    degparts = _deg_kernel(col)
    dis_col, g = _scale1_call(degparts, x)
    for r in range(3):
        p0, p1 = _prop_kernel(g, packed3, zeros_rows)
        if r < 2:
            g = _scale2_call(dis_col, p0, p1, g)
        else:
            out = _final_call(dis_col, p0, p1, g)
    return out.reshape(())
